# baseline (speedup 1.0000x reference)
import jax
import jax.numpy as jnp
import numpy as np
from jax.experimental import pallas as pl
from jax.experimental.pallas import tpu as pltpu

_SCALE = 1.0 / 0.07
_QUEUE = 4096


def _noise_constant() -> np.ndarray:
    def rotl(x, r):
        return (x << np.uint32(r)) | (x >> np.uint32(32 - r))

    k0, k1 = np.uint32(0), np.uint32(1)
    ks = [k0, k1, np.uint32(k0 ^ k1 ^ np.uint32(0x1BD11BDA))]
    x0 = np.zeros(2 * _QUEUE, np.uint32) + ks[0]
    x1 = np.arange(2 * _QUEUE, dtype=np.uint32) + ks[1]
    rotations = [[13, 15, 26, 6], [17, 29, 16, 24]]
    for i in range(5):
        for r in rotations[i % 2]:
            x0 = x0 + x1
            x1 = rotl(x1, r)
            x1 = x1 ^ x0
        x0 = x0 + ks[(i + 1) % 3]
        x1 = x1 + ks[(i + 2) % 3] + np.uint32(i + 1)
    bits = x0 ^ x1
    fl = ((bits >> np.uint32(9)) | np.uint32(0x3F800000)).view(np.float32)
    fl = fl - np.float32(1.0)
    lo = np.float32(np.nextafter(np.float32(-1.0), np.float32(0.0)))
    u = np.maximum(lo, fl * (np.float32(1.0) - lo) + lo)
    w = (-np.log1p(-(u.astype(np.float64) ** 2))).astype(np.float32)
    ws = w - np.float32(2.5)
    wl = np.sqrt(w) - np.float32(3.0)
    cs = [2.81022636e-08, 3.43273939e-07, -3.5233877e-06, -4.39150654e-06,
          0.00021858087, -0.00125372503, -0.00417768164, 0.246640727, 1.50140941]
    cl = [-0.000200214257, 0.000100950558, 0.00134934322, -0.00367342844,
          0.00573950773, -0.0076224613, 0.00943887047, 1.00167406, 2.83297682]
    ps = np.full_like(w, np.float32(cs[0]))
    for c in cs[1:]:
        ps = ps * ws + np.float32(c)
    pl_ = np.full_like(w, np.float32(cl[0]))
    for c in cl[1:]:
        pl_ = pl_ * wl + np.float32(c)
    z = np.where(w < np.float32(5.0), ps, pl_) * u
    z = np.float32(np.sqrt(2.0)) * z
    return (z * np.float32(2500.0 / 111320.0)).reshape(_QUEUE, 2).astype(np.float32)


_NOISE_T = np.ascontiguousarray(_noise_constant().T)


def _loss_kernel(imgs_hbm, gps_hbm, gq_hbm, noiset_hbm, wimg_hbm, w1_hbm,
                 w2_hbm, bs_ref, out_ref,
                 imgs_ref, gps_ref, gq_ref, noiset_ref, w_img_ref, w1_ref,
                 w2_ref, sem):
    bf = jnp.bfloat16
    f32 = jnp.float32
    c00 = (((0,), (0,)), ((), ()))
    c01 = (((0,), (1,)), ((), ()))

    pairs = [(gq_hbm, gq_ref), (imgs_hbm, imgs_ref), (wimg_hbm, w_img_ref),
             (w2_hbm, w2_ref), (w1_hbm, w1_ref), (gps_hbm, gps_ref),
             (noiset_hbm, noiset_ref)]
    cps = [pltpu.make_async_copy(s, d, sem.at[i])
           for i, (s, d) in enumerate(pairs)]
    for c in cps:
        c.start()
    cp_gq, cp_imgs, cp_wimg, cp_w2, cp_w1, cp_gps, cp_noiset = cps

    cp_w1.wait()
    cp_gps.wait()
    w1b = w1_ref[...].astype(bf)
    ht_b = jax.lax.dot_general(w1b, gps_ref[...].astype(bf), c01,
                               preferred_element_type=f32)

    cp_imgs.wait()
    cp_wimg.wait()
    img = jnp.dot(imgs_ref[...].astype(bf), w_img_ref[...].astype(bf),
                  preferred_element_type=f32).astype(bf)
    rn = jnp.dot(img * img, jnp.ones((512, 1), bf), preferred_element_type=f32)

    cp_gq.wait()
    cp_noiset.wait()
    ht_q = jax.lax.dot_general(w1b, gq_ref[...].astype(bf), c01,
                               preferred_element_type=f32)
    ht_q = ht_q + jax.lax.dot_general(w1b, noiset_ref[...].astype(bf), c00,
                                      preferred_element_type=f32)
    ht = jnp.concatenate([ht_b, ht_q], axis=1)
    ht = jnp.maximum(ht, 0).astype(bf)

    cp_w2.wait()
    loct = jax.lax.dot_general(w2_ref[...].astype(bf), ht, c00,
                               preferred_element_type=f32).astype(bf)
    cn = jnp.dot(jnp.ones((1, 512), bf), loct * loct,
                 preferred_element_type=f32)

    raw = jnp.dot(img, loct, preferred_element_type=f32)
    a = jax.lax.rsqrt(rn) * _SCALE
    b = jax.lax.rsqrt(cn)
    arg = raw * a * b
    lse = jnp.log(jnp.sum(jnp.exp(arg), axis=-1))

    bsz = arg.shape[0]
    eye = (jax.lax.broadcasted_iota(jnp.int32, (bsz, bsz), 0)
           == jax.lax.broadcasted_iota(jnp.int32, (bsz, bsz), 1))
    diag_sum = jnp.sum(jnp.where(eye, arg[:, :bsz], 0.0))

    bs = bs_ref[0, 0].astype(f32)
    out_ref[...] = ((jnp.sum(lse) - diag_sum) / bs).reshape(1, 1)


@jax.jit
def _run(imgs, gps, gps_queue, w_img, w1, w2, bs):
    f32 = jnp.float32
    hbm = pl.BlockSpec(memory_space=pl.ANY)
    out = pl.pallas_call(
        _loss_kernel,
        out_shape=jax.ShapeDtypeStruct((1, 1), jnp.float32),
        in_specs=[hbm] * 7 + [pl.BlockSpec(memory_space=pltpu.SMEM)],
        out_specs=pl.BlockSpec(memory_space=pltpu.VMEM),
        scratch_shapes=[
            pltpu.VMEM((512, 768), f32),
            pltpu.VMEM((512, 2), f32),
            pltpu.VMEM((4096, 2), f32),
            pltpu.VMEM((2, 4096), f32),
            pltpu.VMEM((768, 512), f32),
            pltpu.VMEM((2, 512), f32),
            pltpu.VMEM((512, 512), f32),
            pltpu.SemaphoreType.DMA((7,)),
        ],
        compiler_params=pltpu.CompilerParams(
            vmem_limit_bytes=100 * 1024 * 1024,
        ),
    )(imgs, gps, gps_queue, _NOISE_T, w_img, w1, w2, bs)
    return out[0, 0]


def kernel(imgs, gps, gps_queue, W_img, W1, W2, batch_size):
    bs = jnp.asarray(batch_size, jnp.int32).reshape(1, 1)
    return _run(imgs, gps, gps_queue, W_img, W1, W2, bs)


# speedup vs baseline: 1.3795x; 1.3795x over previous
import jax
import jax.numpy as jnp
import numpy as np
from jax.experimental import pallas as pl
from jax.experimental.pallas import tpu as pltpu

_SCALE = 1.0 / 0.07
_QUEUE = 4096


def _noise_constant() -> np.ndarray:
    def rotl(x, r):
        return (x << np.uint32(r)) | (x >> np.uint32(32 - r))

    k0, k1 = np.uint32(0), np.uint32(1)
    ks = [k0, k1, np.uint32(k0 ^ k1 ^ np.uint32(0x1BD11BDA))]
    x0 = np.zeros(2 * _QUEUE, np.uint32) + ks[0]
    x1 = np.arange(2 * _QUEUE, dtype=np.uint32) + ks[1]
    rotations = [[13, 15, 26, 6], [17, 29, 16, 24]]
    for i in range(5):
        for r in rotations[i % 2]:
            x0 = x0 + x1
            x1 = rotl(x1, r)
            x1 = x1 ^ x0
        x0 = x0 + ks[(i + 1) % 3]
        x1 = x1 + ks[(i + 2) % 3] + np.uint32(i + 1)
    bits = x0 ^ x1
    fl = ((bits >> np.uint32(9)) | np.uint32(0x3F800000)).view(np.float32)
    fl = fl - np.float32(1.0)
    lo = np.float32(np.nextafter(np.float32(-1.0), np.float32(0.0)))
    u = np.maximum(lo, fl * (np.float32(1.0) - lo) + lo)
    w = (-np.log1p(-(u.astype(np.float64) ** 2))).astype(np.float32)
    ws = w - np.float32(2.5)
    wl = np.sqrt(w) - np.float32(3.0)
    cs = [2.81022636e-08, 3.43273939e-07, -3.5233877e-06, -4.39150654e-06,
          0.00021858087, -0.00125372503, -0.00417768164, 0.246640727, 1.50140941]
    cl = [-0.000200214257, 0.000100950558, 0.00134934322, -0.00367342844,
          0.00573950773, -0.0076224613, 0.00943887047, 1.00167406, 2.83297682]
    ps = np.full_like(w, np.float32(cs[0]))
    for c in cs[1:]:
        ps = ps * ws + np.float32(c)
    pl_ = np.full_like(w, np.float32(cl[0]))
    for c in cl[1:]:
        pl_ = pl_ * wl + np.float32(c)
    z = np.where(w < np.float32(5.0), ps, pl_) * u
    z = np.float32(np.sqrt(2.0)) * z
    return (z * np.float32(2500.0 / 111320.0)).reshape(_QUEUE, 2).astype(np.float32)


_NOISE = _noise_constant()


def _loss_kernel(imgs_ref, gallt_ref, w_img_ref, w1_ref, w2_ref, bs_ref,
                 out_ref):
    bf = jnp.bfloat16
    f32 = jnp.float32
    c00 = (((0,), (0,)), ((), ()))

    ht = jax.lax.dot_general(w1_ref[...].astype(bf), gallt_ref[...].astype(bf),
                             c00, preferred_element_type=f32)
    ht = jnp.maximum(ht, 0).astype(bf)

    loct = jax.lax.dot_general(w2_ref[...].astype(bf), ht, c00,
                               preferred_element_type=f32).astype(bf)
    cn = jnp.dot(jnp.ones((1, 512), bf), loct * loct,
                 preferred_element_type=f32)

    img = jnp.dot(imgs_ref[...].astype(bf), w_img_ref[...].astype(bf),
                  preferred_element_type=f32).astype(bf)
    rn = jnp.dot(img * img, jnp.ones((512, 1), bf), preferred_element_type=f32)

    raw = jnp.dot(img, loct, preferred_element_type=f32)
    a = jax.lax.rsqrt(rn) * _SCALE
    b = jax.lax.rsqrt(cn)
    arg = raw * a * b
    lse = jnp.log(jnp.sum(jnp.exp(arg), axis=-1))

    bsz = arg.shape[0]
    eye = (jax.lax.broadcasted_iota(jnp.int32, (bsz, bsz), 0)
           == jax.lax.broadcasted_iota(jnp.int32, (bsz, bsz), 1))
    diag_sum = jnp.sum(jnp.where(eye, arg[:, :bsz], 0.0))

    bs = bs_ref[0, 0].astype(f32)
    out_ref[...] = ((jnp.sum(lse) - diag_sum) / bs).reshape(1, 1)


@jax.jit
def _run(imgs, gps, gps_queue, w_img, w1, w2, bs):
    vmem = pl.BlockSpec(memory_space=pltpu.VMEM)
    gallt = jnp.concatenate([gps, gps_queue + _NOISE], axis=0).T
    out = pl.pallas_call(
        _loss_kernel,
        out_shape=jax.ShapeDtypeStruct((1, 1), jnp.float32),
        in_specs=[vmem] * 5 + [pl.BlockSpec(memory_space=pltpu.SMEM)],
        out_specs=vmem,
        compiler_params=pltpu.CompilerParams(
            vmem_limit_bytes=100 * 1024 * 1024,
        ),
    )(imgs, gallt, w_img, w1, w2, bs)
    return out[0, 0]


def kernel(imgs, gps, gps_queue, W_img, W1, W2, batch_size):
    bs = jnp.asarray(batch_size, jnp.int32).reshape(1, 1)
    return _run(imgs, gps, gps_queue, W_img, W1, W2, bs)
